# baseline (speedup 1.0000x reference)
import jax
import jax.numpy as jnp
from jax import lax
from jax.experimental import pallas as pl
from jax.experimental.pallas import tpu as pltpu
from jax.experimental.pallas import tpu_sc as plsc

VOCAB = 1000000
SEQ_LEN = 200
EMBED_DIM = 64
BATCH = 1024

NC, NS, L = 2, 16, 16
NW = NC * NS
IPW = BATCH // NW
IPC = 2
NCH = IPW // IPC
GSZ = 100
SPI = SEQ_LEN // GSZ
NG = IPC * SPI
NSL = 3
SCALE = 8.0


def _body(x_hbm, table_hbm, pos8_hbm, out_hbm, *refs):
    ibufs = refs[0:2]
    gbufs = refs[2:2 + NSL]
    oA, oB = refs[2 + NSL:4 + NSL]
    sps = refs[4 + NSL:4 + 2 * NSL]
    sis = refs[4 + 2 * NSL:6 + 2 * NSL]
    sgs = refs[6 + 2 * NSL:6 + 3 * NSL]
    soA, soB = refs[6 + 3 * NSL:8 + 3 * NSL]

    wid = lax.axis_index("s") * NC + lax.axis_index("c")
    c_base = wid * IPW * SPI

    def pre(j):
        sl = j % NSL
        g = gbufs[sl]
        ibuf = ibufs[j % 2]
        pltpu.async_copy(pos8_hbm, g, sps[sl])
        pltpu.async_copy(x_hbm.at[pl.ds(c_base + j * NG, NG)], ibuf,
                         sis[j % 2])

    def gather(j):
        sl = j % NSL
        g = gbufs[sl]
        ibuf = ibufs[j % 2]
        pltpu.make_async_copy(pos8_hbm, g, sps[sl]).wait()
        pltpu.make_async_copy(x_hbm.at[pl.ds(0, NG)], ibuf,
                              sis[j % 2]).wait()
        for k in range(NG):
            pltpu.async_copy(
                table_hbm.at[ibuf.at[k]], g.at[k], sgs[sl], add=True)

    def consume(j):
        sl = j % NSL
        g = gbufs[sl]
        ibuf = ibufs[j % 2]
        for k in range(NG):
            pltpu.make_async_copy(
                table_hbm.at[ibuf.at[k]], g.at[k], sgs[sl]).wait()
        cf0 = (c_base + j * NG) * (GSZ * EMBED_DIM // 128)
        for half, (o, so) in enumerate(((oA, soA), (oB, soB))):
            if j >= 1:
                pltpu.make_async_copy(o, out_hbm.at[pl.ds(0, GSZ)],
                                      so).wait()

            def row_body(rp, carry, _half=half, _o=o):
                for kk in range(NG // 2):
                    k = _half * (NG // 2) + kk
                    row = kk * (GSZ // 2) + rp
                    for par in range(2):
                        r = 2 * rp + par
                        for q in range(EMBED_DIM // L):
                            qs = pl.ds(q * L, L)
                            os_ = pl.ds(par * EMBED_DIM + q * L, L)
                            _o[row, os_] = g[k, r, qs] * SCALE
                return carry

            lax.fori_loop(0, GSZ // 2, row_body, 0, unroll=2)
            pltpu.async_copy(
                o, out_hbm.at[pl.ds(cf0 + half * GSZ, GSZ)], so)

    for j in range(NCH + 2):
        if j < NCH:
            pre(j)
        if 1 <= j <= NCH:
            gather(j - 1)
        if j >= 2:
            consume(j - 2)

    for o, so in ((oA, soA), (oB, soB)):
        pltpu.make_async_copy(o, out_hbm.at[pl.ds(0, GSZ)], so).wait()


@jax.jit
def kernel(x, token_table, pos_table):
    x2 = x.reshape(BATCH * SEQ_LEN // GSZ, GSZ)
    pos8 = (pos_table * (1.0 / SCALE)).reshape(SPI, GSZ, EMBED_DIM)
    pos8t = jnp.tile(pos8, (IPC, 1, 1))
    mesh = plsc.VectorSubcoreMesh(
        core_axis_name="c", subcore_axis_name="s",
        num_cores=NC, num_subcores=NS)
    scratch = (
        [pltpu.VMEM((NG, GSZ), jnp.int32)] * 2
        + [pltpu.VMEM((NG, GSZ, EMBED_DIM), jnp.float32)] * NSL
        + [pltpu.VMEM((GSZ, 128), jnp.float32)] * 2
        + [pltpu.SemaphoreType.DMA] * (NSL + 2 + NSL + 2)
    )
    out = pl.kernel(
        _body,
        out_type=jax.ShapeDtypeStruct(
            (BATCH * SEQ_LEN * EMBED_DIM // 128, 128), jnp.float32),
        mesh=mesh,
        scratch_types=scratch,
        compiler_params=pltpu.CompilerParams(use_tc_tiling_on_sc=False),
    )(x2, token_table, pos8t)
    return out.reshape(BATCH, SEQ_LEN, EMBED_DIM)


# speedup vs baseline: 1.0014x; 1.0014x over previous
import jax
import jax.numpy as jnp
from jax import lax
from jax.experimental import pallas as pl
from jax.experimental.pallas import tpu as pltpu
from jax.experimental.pallas import tpu_sc as plsc

VOCAB = 1000000
SEQ_LEN = 200
EMBED_DIM = 64
BATCH = 1024

NC, NS, L = 2, 16, 16
NW = NC * NS
IPW = BATCH // NW
IPC = 2
NCH = IPW // IPC
GSZ = 100
SPI = SEQ_LEN // GSZ
NG = IPC * SPI
NSL = 3
SCALE = 8.0


def _body(x_hbm, table_hbm, pos8_hbm, out_hbm, *refs):
    ibufs = refs[0:NSL]
    gbufs = refs[NSL:2 * NSL]
    oA, oB = refs[2 * NSL:2 * NSL + 2]
    sps = refs[2 * NSL + 2:3 * NSL + 2]
    sis = refs[3 * NSL + 2:4 * NSL + 2]
    sgs = refs[4 * NSL + 2:5 * NSL + 2]
    soA, soB = refs[5 * NSL + 2:5 * NSL + 4]

    wid = lax.axis_index("s") * NC + lax.axis_index("c")
    c_base = wid * IPW * SPI

    def pre(j):
        sl = j % NSL
        g = gbufs[sl]
        ibuf = ibufs[j % NSL]
        pltpu.async_copy(pos8_hbm, g, sps[sl])
        pltpu.async_copy(x_hbm.at[pl.ds(c_base + j * NG, NG)], ibuf,
                         sis[j % NSL])

    def gather(j):
        sl = j % NSL
        g = gbufs[sl]
        ibuf = ibufs[j % NSL]
        pltpu.make_async_copy(pos8_hbm, g, sps[sl]).wait()
        pltpu.make_async_copy(x_hbm.at[pl.ds(0, NG)], ibuf,
                              sis[j % NSL]).wait()
        for k in range(NG):
            pltpu.async_copy(
                table_hbm.at[ibuf.at[k]], g.at[k], sgs[sl], add=True)

    def consume(j):
        sl = j % NSL
        g = gbufs[sl]
        ibuf = ibufs[j % NSL]
        for k in range(NG):
            pltpu.make_async_copy(
                table_hbm.at[ibuf.at[k]], g.at[k], sgs[sl]).wait()
        cf0 = (c_base + j * NG) * (GSZ * EMBED_DIM // 128)
        for half, (o, so) in enumerate(((oA, soA), (oB, soB))):
            if j >= 1:
                pltpu.make_async_copy(o, out_hbm.at[pl.ds(0, GSZ)],
                                      so).wait()

            def row_body(rp, carry, _half=half, _o=o):
                for kk in range(NG // 2):
                    k = _half * (NG // 2) + kk
                    row = kk * (GSZ // 2) + rp
                    for par in range(2):
                        r = 2 * rp + par
                        for q in range(EMBED_DIM // L):
                            qs = pl.ds(q * L, L)
                            os_ = pl.ds(par * EMBED_DIM + q * L, L)
                            _o[row, os_] = g[k, r, qs] * SCALE
                return carry

            lax.fori_loop(0, GSZ // 2, row_body, 0, unroll=2)
            pltpu.async_copy(
                o, out_hbm.at[pl.ds(cf0 + half * GSZ, GSZ)], so)

    for j in range(NCH + 2):
        if j < NCH:
            pre(j)
        if 1 <= j <= NCH:
            gather(j - 1)
        if j >= 2:
            consume(j - 2)

    for o, so in ((oA, soA), (oB, soB)):
        pltpu.make_async_copy(o, out_hbm.at[pl.ds(0, GSZ)], so).wait()


@jax.jit
def kernel(x, token_table, pos_table):
    x2 = x.reshape(BATCH * SEQ_LEN // GSZ, GSZ)
    pos8 = (pos_table * (1.0 / SCALE)).reshape(SPI, GSZ, EMBED_DIM)
    pos8t = jnp.tile(pos8, (IPC, 1, 1))
    mesh = plsc.VectorSubcoreMesh(
        core_axis_name="c", subcore_axis_name="s",
        num_cores=NC, num_subcores=NS)
    scratch = (
        [pltpu.VMEM((NG, GSZ), jnp.int32)] * NSL
        + [pltpu.VMEM((NG, GSZ, EMBED_DIM), jnp.float32)] * NSL
        + [pltpu.VMEM((GSZ, 128), jnp.float32)] * 2
        + [pltpu.SemaphoreType.DMA] * (3 * NSL + 2)
    )
    out = pl.kernel(
        _body,
        out_type=jax.ShapeDtypeStruct(
            (BATCH * SEQ_LEN * EMBED_DIM // 128, 128), jnp.float32),
        mesh=mesh,
        scratch_types=scratch,
        compiler_params=pltpu.CompilerParams(use_tc_tiling_on_sc=False),
    )(x2, token_table, pos8t)
    return out.reshape(BATCH, SEQ_LEN, EMBED_DIM)


# speedup vs baseline: 1.0124x; 1.0110x over previous
import jax
import jax.numpy as jnp
from jax import lax
from jax.experimental import pallas as pl
from jax.experimental.pallas import tpu as pltpu
from jax.experimental.pallas import tpu_sc as plsc

VOCAB = 1000000
SEQ_LEN = 200
EMBED_DIM = 64
BATCH = 1024

NC, NS, L = 2, 16, 16
NW = NC * NS
IPW = BATCH // NW
IPC = 2
NCH = IPW // IPC
GSZ = 100
SPI = SEQ_LEN // GSZ
NG = IPC * SPI
NSL = 3
SCALE = 8.0


def _body(x_hbm, table_hbm, pos8_hbm, out_hbm, *refs):
    ibufs = refs[0:NSL]
    gbufs = refs[NSL:2 * NSL]
    sps = refs[2 * NSL:3 * NSL]
    sis = refs[3 * NSL:4 * NSL]
    sgs = refs[4 * NSL:5 * NSL]
    sos = refs[5 * NSL:6 * NSL]

    wid = lax.axis_index("s") * NC + lax.axis_index("c")
    c_base = wid * IPW * SPI

    def pre(j):
        sl = j % NSL
        ibuf, g = ibufs[sl], gbufs[sl]
        if j >= NSL:
            pltpu.make_async_copy(g, out_hbm.at[pl.ds(0, NG)],
                                  sos[sl]).wait()
        pltpu.async_copy(pos8_hbm, g, sps[sl])
        pltpu.async_copy(x_hbm.at[pl.ds(c_base + j * NG, NG)], ibuf,
                         sis[sl])

    def gather(j):
        sl = j % NSL
        ibuf, g = ibufs[sl], gbufs[sl]
        pltpu.make_async_copy(pos8_hbm, g, sps[sl]).wait()
        pltpu.make_async_copy(x_hbm.at[pl.ds(0, NG)], ibuf, sis[sl]).wait()
        for k in range(NG):
            pltpu.async_copy(
                table_hbm.at[ibuf.at[k]], g.at[k], sgs[sl], add=True)

    def consume(j):
        sl = j % NSL
        ibuf, g = ibufs[sl], gbufs[sl]
        for k in range(NG):
            pltpu.make_async_copy(
                table_hbm.at[ibuf.at[k]], g.at[k], sgs[sl]).wait()

        def row_body(r, carry):
            for k in range(NG):
                for q in range(EMBED_DIM // L):
                    qs = pl.ds(q * L, L)
                    g[k, r, qs] = g[k, r, qs] * SCALE
            return carry

        lax.fori_loop(0, GSZ, row_body, 0, unroll=2)
        pltpu.async_copy(g, out_hbm.at[pl.ds(c_base + j * NG, NG)],
                         sos[sl])

    for j in range(NCH + 2):
        if j < NCH:
            pre(j)
        if 1 <= j <= NCH:
            gather(j - 1)
        if j >= 2:
            consume(j - 2)

    for j in range(NCH - NSL, NCH):
        sl = j % NSL
        pltpu.make_async_copy(gbufs[sl], out_hbm.at[pl.ds(0, NG)],
                              sos[sl]).wait()


@jax.jit
def kernel(x, token_table, pos_table):
    x2 = x.reshape(BATCH * SEQ_LEN // GSZ, GSZ)
    pos8 = (pos_table * (1.0 / SCALE)).reshape(SPI, GSZ, EMBED_DIM)
    pos8t = jnp.tile(pos8, (IPC, 1, 1))
    mesh = plsc.VectorSubcoreMesh(
        core_axis_name="c", subcore_axis_name="s",
        num_cores=NC, num_subcores=NS)
    scratch = (
        [pltpu.VMEM((NG, GSZ), jnp.int32)] * NSL
        + [pltpu.VMEM((NG, GSZ, EMBED_DIM), jnp.float32)] * NSL
        + [pltpu.SemaphoreType.DMA] * (4 * NSL)
    )
    out = pl.kernel(
        _body,
        out_type=jax.ShapeDtypeStruct(
            (BATCH * SEQ_LEN // GSZ, GSZ, EMBED_DIM), jnp.float32),
        mesh=mesh,
        scratch_types=scratch,
        compiler_params=pltpu.CompilerParams(use_tc_tiling_on_sc=False),
    )(x2, token_table, pos8t)
    return out.reshape(BATCH, SEQ_LEN, EMBED_DIM)
